# baseline (speedup 1.0000x reference)
import functools

import jax
import jax.numpy as jnp
from jax import lax
from jax.experimental import pallas as pl
from jax.experimental.pallas import tpu as pltpu
from jax.experimental.pallas import tpu_sc as plsc

_MAX_LEN = 8192
_D = 1024
_B = 4
_NC = 2
_NS = 16
_NW = _NC * _NS
_ROWS = _MAX_LEN // _NW
_CHUNK = 32
_NCHUNK = _ROWS // _CHUNK

_mesh = plsc.VectorSubcoreMesh(core_axis_name="c", subcore_axis_name="s")


@functools.partial(
    pl.kernel,
    mesh=_mesh,
    out_type=jax.ShapeDtypeStruct((_B * _MAX_LEN, _D), jnp.float32),
    scratch_types=[
        pltpu.VMEM((2, _CHUNK, _D), jnp.float32),
        pltpu.SemaphoreType.DMA,
        pltpu.SemaphoreType.DMA,
    ],
)
def _bcast(pe_hbm, out_hbm, buf, rsem, wsem):
    wid = lax.axis_index("s") * _NC + lax.axis_index("c")
    base = wid * _ROWS

    def rd(i, slot):
        return pltpu.make_async_copy(
            pe_hbm.at[pl.ds(base + i * _CHUNK, _CHUNK)], buf.at[slot], rsem
        )

    def wr(i, slot, b):
        return pltpu.make_async_copy(
            buf.at[slot],
            out_hbm.at[pl.ds(b * _MAX_LEN + base + i * _CHUNK, _CHUNK)],
            wsem,
        )

    rd(0, 0).start()
    for i in range(_NCHUNK):
        slot = i % 2
        rd(i, slot).wait()
        for b in range(_B):
            wr(i, slot, b).start()
        if i + 1 < _NCHUNK:
            if i >= 1:
                for b in range(_B):
                    wr(i - 1, (i - 1) % 2, b).wait()
            rd(i + 1, (i + 1) % 2).start()
    for i in (_NCHUNK - 2, _NCHUNK - 1):
        for b in range(_B):
            wr(i, i % 2, b).wait()


def kernel(x, pe_table):
    del x
    out = _bcast(pe_table)
    return out.reshape(_B, _MAX_LEN, _D)
